# baseline (speedup 1.0000x reference)
import functools

import jax
import jax.numpy as jnp
from jax.experimental import pallas as pl
from jax.experimental.pallas import tpu as pltpu

_LANE = 128


def _round_up(x, m):
    return ((x + m - 1) // m) * m


def _conv_one(x_b, w_flat, kernel_size, dilation, pad, tl):
    c_in = x_b.shape[0]
    z = jnp.zeros((c_in, _LANE), jnp.bfloat16)
    xc = jnp.concatenate([z, x_b.astype(jnp.bfloat16), z], axis=1)
    taps = [
        jax.lax.slice_in_dim(xc, _LANE - pad + k * dilation,
                             _LANE - pad + k * dilation + tl, axis=1)
        for k in range(kernel_size)
    ]
    xs = jnp.concatenate(taps, axis=0)
    return jax.lax.dot_general(
        w_flat, xs,
        dimension_numbers=(((1,), (0,)), ((), ())),
        preferred_element_type=jnp.float32)


def _stats_kernel(x_ref, w_ref, stats_ref, *, kernel_size, dilation, pad, tl,
                  n_blk, l_out, c_out):
    w_flat = w_ref[...]
    acc1 = jnp.zeros((c_out, tl), jnp.float32)
    acc2 = jnp.zeros((c_out, tl), jnp.float32)
    for b in range(n_blk):
        conv = _conv_one(x_ref[b], w_flat, kernel_size, dilation, pad, tl)
        acc1 = acc1 + conv
        acc2 = acc2 + conv * conv
    if l_out != tl:
        lane = jax.lax.broadcasted_iota(jnp.int32, (c_out, tl), 1)
        mask = (lane < l_out).astype(jnp.float32)
        acc1 = acc1 * mask
        acc2 = acc2 * mask
    s1 = jnp.sum(acc1, axis=1, keepdims=True)
    s2 = jnp.sum(acc2, axis=1, keepdims=True)
    stats_ref[...] = jnp.concatenate([s1, s2], axis=1)


def _apply_kernel(x_ref, w_ref, stats_ref, g_ref, b_ref, out_ref, *,
                  kernel_size, dilation, pad, tl, n_blk, cnt, eps):
    st = jnp.sum(stats_ref[...], axis=0)
    inv_cnt = jnp.float32(1.0 / cnt)
    mean = st[:, 0:1] * inv_cnt
    var = jnp.maximum(st[:, 1:2] * inv_cnt - mean * mean, 0.0)
    scale = g_ref[...] * jax.lax.rsqrt(var + eps)
    shift = b_ref[...] - mean * scale
    w_flat = w_ref[...]
    for b in range(n_blk):
        conv = _conv_one(x_ref[b], w_flat, kernel_size, dilation, pad, tl)
        out_ref[b] = jnp.maximum(conv * scale + shift, 0.0)


def kernel(x, weight, bias, gamma, beta):
    del bias
    kernel_size = weight.shape[2]
    dilation = 1
    eps = 1e-3

    n, c_in, length = x.shape
    c_out = weight.shape[0]
    pad = (dilation * (kernel_size - 1)) // 2
    halo = dilation * (kernel_size - 1)
    l_out = length + 2 * pad - halo
    assert halo < _LANE and pad < _LANE
    tl = _round_up(l_out, _LANE)
    assert tl <= length + _LANE

    w_flat = jnp.transpose(weight, (0, 2, 1)).reshape(
        c_out, kernel_size * c_in).astype(jnp.bfloat16)
    g2 = gamma.astype(jnp.float32).reshape(c_out, 1)
    b2 = beta.astype(jnp.float32).reshape(c_out, 1)

    n_blk = 4
    while n % n_blk:
        n_blk //= 2
    n_blocks = n // n_blk
    grid = (n_blocks,)

    x_spec = pl.BlockSpec((n_blk, c_in, length), lambda i: (i, 0, 0))
    w_spec = pl.BlockSpec((c_out, kernel_size * c_in), lambda i: (0, 0))
    vmem_limit = 64 * 1024 * 1024

    stats_parts = pl.pallas_call(
        functools.partial(_stats_kernel, kernel_size=kernel_size,
                          dilation=dilation, pad=pad, tl=tl, n_blk=n_blk,
                          l_out=l_out, c_out=c_out),
        out_shape=jax.ShapeDtypeStruct((n_blocks, c_out, 2), jnp.float32),
        grid=grid,
        in_specs=[x_spec, w_spec],
        out_specs=pl.BlockSpec((None, c_out, 2), lambda i: (i, 0, 0)),
        compiler_params=pltpu.CompilerParams(
            dimension_semantics=("parallel",),
            vmem_limit_bytes=vmem_limit),
    )(x, w_flat)

    return pl.pallas_call(
        functools.partial(_apply_kernel, kernel_size=kernel_size,
                          dilation=dilation, pad=pad, tl=tl, n_blk=n_blk,
                          cnt=float(n * l_out), eps=eps),
        out_shape=jax.ShapeDtypeStruct((n, c_out, l_out), jnp.float32),
        grid=grid,
        in_specs=[x_spec, w_spec,
                  pl.BlockSpec((n_blocks, c_out, 2), lambda i: (0, 0, 0)),
                  pl.BlockSpec((c_out, 1), lambda i: (0, 0)),
                  pl.BlockSpec((c_out, 1), lambda i: (0, 0))],
        out_specs=pl.BlockSpec((n_blk, c_out, tl), lambda i: (i, 0, 0)),
        compiler_params=pltpu.CompilerParams(
            dimension_semantics=("parallel",),
            vmem_limit_bytes=vmem_limit),
    )(x, w_flat, stats_parts, g2, b2)


# speedup vs baseline: 1.9652x; 1.9652x over previous
import functools

import jax
import jax.numpy as jnp
from jax.experimental import pallas as pl
from jax.experimental.pallas import tpu as pltpu

_LANE = 128
_L_BLK = 64
_HALO_BLK = 8


def _cdiv(a, b):
    return -(-a // b)


def _conv_tile(x_ref, xh_ref, w_ref, kernel_size, l_blk, n_total, c_in):
    xc = jnp.concatenate([x_ref[...], xh_ref[...]], axis=0)
    taps = [
        jax.lax.slice_in_dim(xc, k, k + l_blk, axis=0)
        for k in range(kernel_size)
    ]
    xs = jnp.concatenate(taps, axis=2)
    xs = xs.reshape(l_blk * n_total, kernel_size * c_in)
    return jax.lax.dot_general(
        xs, w_ref[...],
        dimension_numbers=(((1,), (0,)), ((), ())),
        preferred_element_type=jnp.float32)


def _stats_kernel(x_ref, xh_ref, w_ref, stats_ref, *, kernel_size, l_blk,
                  n_total, c_in, l_out):
    t = pl.program_id(0)
    y = _conv_tile(x_ref, xh_ref, w_ref, kernel_size, l_blk, n_total, c_in)
    li = jax.lax.broadcasted_iota(jnp.int32, (l_blk, n_total, 1), 0)
    valid = (li + t * l_blk < l_out).astype(jnp.float32)
    y = y * valid.reshape(l_blk * n_total, 1)
    s1 = jnp.sum(y, axis=0, keepdims=True)
    s2 = jnp.sum(y * y, axis=0, keepdims=True)
    stats_ref[...] = jnp.concatenate([s1, s2], axis=0)


def _apply_kernel(x_ref, xh_ref, w_ref, stats_ref, g_ref, b_ref, out_ref, *,
                  kernel_size, l_blk, n_total, c_in, cnt, eps):
    st = jnp.sum(stats_ref[...], axis=0)
    inv_cnt = jnp.float32(1.0 / cnt)
    mean = st[0:1, :] * inv_cnt
    var = jnp.maximum(st[1:2, :] * inv_cnt - mean * mean, 0.0)
    scale = g_ref[...] * jax.lax.rsqrt(var + eps)
    shift = b_ref[...] - mean * scale
    y = _conv_tile(x_ref, xh_ref, w_ref, kernel_size, l_blk, n_total, c_in)
    y = jnp.maximum(y * scale + shift, 0.0)
    out_ref[...] = y.reshape(l_blk, n_total, -1)


def kernel(x, weight, bias, gamma, beta):
    del bias
    kernel_size = weight.shape[2]
    dilation = 1
    eps = 1e-3

    n, c_in, length = x.shape
    c_out = weight.shape[0]
    pad = (dilation * (kernel_size - 1)) // 2
    halo = dilation * (kernel_size - 1)
    l_out = length + 2 * pad - halo
    assert halo < _HALO_BLK + pad and n % 8 == 0

    n_tiles = _cdiv(l_out, _L_BLK)
    rows = n_tiles * _L_BLK + _HALO_BLK
    xp = jnp.pad(
        jnp.transpose(x, (2, 0, 1)).astype(jnp.bfloat16),
        ((pad, rows - pad - length), (0, 0), (0, 0)))
    w_t = jnp.transpose(weight, (2, 1, 0)).reshape(
        kernel_size * c_in, c_out).astype(jnp.bfloat16)
    g2 = gamma.astype(jnp.float32).reshape(1, c_out)
    b2 = beta.astype(jnp.float32).reshape(1, c_out)

    grid = (n_tiles,)
    units = _L_BLK // _HALO_BLK
    x_spec = pl.BlockSpec((_L_BLK, n, c_in), lambda t: (t, 0, 0))
    xh_spec = pl.BlockSpec((_HALO_BLK, n, c_in),
                           lambda t: (t * units + units, 0, 0))
    w_spec = pl.BlockSpec((kernel_size * c_in, c_out), lambda t: (0, 0))
    vmem_limit = 64 * 1024 * 1024

    stats_parts = pl.pallas_call(
        functools.partial(_stats_kernel, kernel_size=kernel_size,
                          l_blk=_L_BLK, n_total=n, c_in=c_in, l_out=l_out),
        out_shape=jax.ShapeDtypeStruct((n_tiles, 2, c_out), jnp.float32),
        grid=grid,
        in_specs=[x_spec, xh_spec, w_spec],
        out_specs=pl.BlockSpec((None, 2, c_out), lambda t: (t, 0, 0)),
        compiler_params=pltpu.CompilerParams(
            dimension_semantics=("parallel",),
            vmem_limit_bytes=vmem_limit),
    )(xp, xp, w_t)

    out_t = pl.pallas_call(
        functools.partial(_apply_kernel, kernel_size=kernel_size,
                          l_blk=_L_BLK, n_total=n, c_in=c_in,
                          cnt=float(n * l_out), eps=eps),
        out_shape=jax.ShapeDtypeStruct((l_out, n, c_out), jnp.float32),
        grid=grid,
        in_specs=[x_spec, xh_spec, w_spec,
                  pl.BlockSpec((n_tiles, 2, c_out), lambda t: (0, 0, 0)),
                  pl.BlockSpec((1, c_out), lambda t: (0, 0)),
                  pl.BlockSpec((1, c_out), lambda t: (0, 0))],
        out_specs=pl.BlockSpec((_L_BLK, n, c_out), lambda t: (t, 0, 0)),
        compiler_params=pltpu.CompilerParams(
            dimension_semantics=("parallel",),
            vmem_limit_bytes=vmem_limit),
    )(xp, xp, w_t, stats_parts, g2, b2)

    return jnp.transpose(out_t, (1, 2, 0))


# speedup vs baseline: 2.2436x; 1.1416x over previous
import functools

import jax
import jax.numpy as jnp
from jax.experimental import pallas as pl
from jax.experimental.pallas import tpu as pltpu

_L_BLK = 128
_HALO_BLK = 8


def _cdiv(a, b):
    return -(-a // b)


def _conv_tile(pre_ref, x_ref, post_ref, w_ref, kernel_size, pad, l_blk,
               n_total, c_in, n_tiles):
    t = pl.program_id(0)
    lead = pad
    trail = kernel_size - 1 - pad
    pre = jnp.where(t > 0, pre_ref[_HALO_BLK - lead:, :, :], 0)
    post = jnp.where(t < n_tiles - 1, post_ref[:trail, :, :], 0)
    xc = jnp.concatenate([pre, x_ref[...], post], axis=0)
    taps = [
        jax.lax.slice_in_dim(xc, k, k + l_blk, axis=0)
        for k in range(kernel_size)
    ]
    xs = jnp.concatenate(taps, axis=2)
    xs = xs.reshape(l_blk * n_total, kernel_size * c_in)
    return jax.lax.dot_general(
        xs, w_ref[...],
        dimension_numbers=(((1,), (0,)), ((), ())),
        preferred_element_type=jnp.float32)


def _stats_kernel(pre_ref, x_ref, post_ref, w_ref, stats_ref, *, kernel_size,
                  pad, l_blk, n_total, c_in, l_out, n_tiles):
    t = pl.program_id(0)
    y = _conv_tile(pre_ref, x_ref, post_ref, w_ref, kernel_size, pad, l_blk,
                   n_total, c_in, n_tiles)
    li = jax.lax.broadcasted_iota(jnp.int32, (l_blk, n_total, 1), 0)
    valid = (li + t * l_blk < l_out).astype(jnp.float32)
    y = y * valid.reshape(l_blk * n_total, 1)
    s1 = jnp.sum(y, axis=0, keepdims=True)
    s2 = jnp.sum(y * y, axis=0, keepdims=True)
    stats_ref[...] = jnp.concatenate([s1, s2], axis=0)


def _apply_kernel(pre_ref, x_ref, post_ref, w_ref, stats_ref, g_ref, b_ref,
                  out_ref, *, kernel_size, pad, l_blk, n_total, c_in, cnt,
                  eps, n_tiles):
    st = jnp.sum(stats_ref[...], axis=0)
    inv_cnt = jnp.float32(1.0 / cnt)
    mean = st[0:1, :] * inv_cnt
    var = jnp.maximum(st[1:2, :] * inv_cnt - mean * mean, 0.0)
    scale = g_ref[...] * jax.lax.rsqrt(var + eps)
    shift = b_ref[...] - mean * scale
    y = _conv_tile(pre_ref, x_ref, post_ref, w_ref, kernel_size, pad, l_blk,
                   n_total, c_in, n_tiles)
    y = jnp.maximum(y * scale + shift, 0.0)
    out_ref[...] = y.reshape(l_blk, n_total, -1)


def kernel(x, weight, bias, gamma, beta):
    del bias
    kernel_size = weight.shape[2]
    dilation = 1
    eps = 1e-3

    n, c_in, length = x.shape
    c_out = weight.shape[0]
    pad = (dilation * (kernel_size - 1)) // 2
    halo = dilation * (kernel_size - 1)
    l_out = length + 2 * pad - halo
    assert halo < _HALO_BLK + pad and pad < _HALO_BLK
    assert n % 8 == 0 and length % _L_BLK == 0

    n_tiles = _cdiv(l_out, _L_BLK)
    units = _L_BLK // _HALO_BLK
    total_units = length // _HALO_BLK

    xt = jnp.transpose(x, (2, 0, 1)).astype(jnp.bfloat16)
    w_t = jnp.transpose(weight, (2, 1, 0)).reshape(
        kernel_size * c_in, c_out).astype(jnp.bfloat16)
    g2 = gamma.astype(jnp.float32).reshape(1, c_out)
    b2 = beta.astype(jnp.float32).reshape(1, c_out)

    grid = (n_tiles,)
    pre_spec = pl.BlockSpec(
        (_HALO_BLK, n, c_in),
        lambda t: (jnp.maximum(t * units - 1, 0), 0, 0))
    x_spec = pl.BlockSpec((_L_BLK, n, c_in), lambda t: (t, 0, 0))
    post_spec = pl.BlockSpec(
        (_HALO_BLK, n, c_in),
        lambda t: (jnp.minimum(t * units + units, total_units - 1), 0, 0))
    w_spec = pl.BlockSpec((kernel_size * c_in, c_out), lambda t: (0, 0))
    vmem_limit = 100 * 1024 * 1024

    stats_parts = pl.pallas_call(
        functools.partial(_stats_kernel, kernel_size=kernel_size, pad=pad,
                          l_blk=_L_BLK, n_total=n, c_in=c_in, l_out=l_out,
                          n_tiles=n_tiles),
        out_shape=jax.ShapeDtypeStruct((n_tiles, 2, c_out), jnp.float32),
        grid=grid,
        in_specs=[pre_spec, x_spec, post_spec, w_spec],
        out_specs=pl.BlockSpec((None, 2, c_out), lambda t: (t, 0, 0)),
        compiler_params=pltpu.CompilerParams(
            dimension_semantics=("parallel",),
            vmem_limit_bytes=vmem_limit),
    )(xt, xt, xt, w_t)

    out_t = pl.pallas_call(
        functools.partial(_apply_kernel, kernel_size=kernel_size, pad=pad,
                          l_blk=_L_BLK, n_total=n, c_in=c_in,
                          cnt=float(n * l_out), eps=eps, n_tiles=n_tiles),
        out_shape=jax.ShapeDtypeStruct((l_out, n, c_out), jnp.float32),
        grid=grid,
        in_specs=[pre_spec, x_spec, post_spec, w_spec,
                  pl.BlockSpec((n_tiles, 2, c_out), lambda t: (0, 0, 0)),
                  pl.BlockSpec((1, c_out), lambda t: (0, 0)),
                  pl.BlockSpec((1, c_out), lambda t: (0, 0))],
        out_specs=pl.BlockSpec((_L_BLK, n, c_out), lambda t: (t, 0, 0)),
        compiler_params=pltpu.CompilerParams(
            dimension_semantics=("parallel",),
            vmem_limit_bytes=vmem_limit),
    )(xt, xt, xt, w_t, stats_parts, g2, b2)

    return jnp.transpose(out_t, (1, 2, 0))
